# baseline (speedup 1.0000x reference)
import functools

import jax
import jax.numpy as jnp
from jax import lax
from jax.experimental import pallas as pl
from jax.experimental.pallas import tpu as pltpu
from jax.experimental.pallas import tpu_sc as plsc

_BB = 4
_HB = _BB // 2
_S = 16
_CH = 32
_KP = 1040


def _matvec_body(d0_ref, d1_ref, c_ref, o_ref):
    for j, d_ref in enumerate((d0_ref, d1_ref)):
        for i in range(_HB):
            bi = j * _HB + i
            d = d_ref[i]
            w = c_ref[bi, :, 0:1024]
            acc = jnp.sum(d * w, axis=1)
            o_ref[bi] = acc[None, :] + c_ref[bi, 0, 1024]


def _tc_matvec(D, c3, nb):
    B, L, K = D.shape
    out = pl.pallas_call(
        _matvec_body,
        grid=(nb // _BB,),
        in_specs=[
            pl.BlockSpec((_HB, L, K), lambda b: (2 * b, 0, 0)),
            pl.BlockSpec((_HB, L, K), lambda b: (2 * b + 1, 0, 0)),
            pl.BlockSpec((_BB, 1, _KP), lambda b: (b, 0, 0)),
        ],
        out_specs=pl.BlockSpec((_BB, 1, L), lambda b: (b, 0, 0)),
        out_shape=jax.ShapeDtypeStruct((nb, 1, L), jnp.float32),
    )(D, D, c3)
    return out.reshape(nb, L, 1)


def _lane_perm(v, idx):
    dn = lax.GatherDimensionNumbers(
        offset_dims=(), collapsed_slice_dims=(0,), start_index_map=(0,))
    return lax.gather(v, idx[:, None], dn, slice_sizes=(1,),
                      mode=lax.GatherScatterMode.PROMISE_IN_BOUNDS)


def _sc_body(nbatch, rows_per_w, d_hbm, c_hbm, out_hbm,
             dbuf, wbuf, bbuf, obuf, sem0, sem1):
    B, L, K = d_hbm.shape
    wid = lax.axis_index("s") * 2 + lax.axis_index("c")
    wpb = L // rows_per_w
    bsc = wid // wpb
    b = (B - nbatch) + bsc
    r0 = (wid % wpb) * rows_per_w

    pltpu.sync_copy(c_hbm.at[b, 0, pl.ds(0, K)], wbuf)
    pltpu.sync_copy(c_hbm.at[b, 0, pl.ds(K, 16)], bbuf)
    lanes = lax.iota(jnp.int32, 16)
    bias = _lane_perm(bbuf[...], jnp.zeros((16,), jnp.int32))

    def _xlane_sum(v):
        for sh in (8, 4, 2, 1):
            v = v + _lane_perm(v, lanes ^ sh)
        return v

    nch = rows_per_w // _CH
    sems = (sem0, sem1)
    pltpu.async_copy(d_hbm.at[b, pl.ds(r0, _CH)], dbuf.at[0], sem0)

    def chunk_pair(cp, carry):
        for par in range(2):
            c = cp * 2 + par
            nxt = jnp.minimum(c + 1, nch - 1)
            pltpu.async_copy(d_hbm.at[b, pl.ds(r0 + nxt * _CH, _CH)],
                             dbuf.at[1 - par], sems[1 - par])
            pltpu.make_async_copy(d_hbm.at[b, pl.ds(r0, _CH)],
                                  dbuf.at[par], sems[par]).wait()

            def row_group(g, _):
                def quad(q, res):
                    row = g * 16 + q * 4
                    a0 = jnp.zeros((16,), jnp.float32)
                    a1 = a0
                    a2 = a0
                    a3 = a0
                    for kc in range(K // 16):
                        off = kc * 16
                        wv = wbuf[pl.ds(off, 16)]
                        a0 = a0 + dbuf[par, row, pl.ds(off, 16)] * wv
                        a1 = a1 + dbuf[par, row + 1, pl.ds(off, 16)] * wv
                        a2 = a2 + dbuf[par, row + 2, pl.ds(off, 16)] * wv
                        a3 = a3 + dbuf[par, row + 3, pl.ds(off, 16)] * wv
                    for t, a in enumerate((a0, a1, a2, a3)):
                        s = _xlane_sum(a)
                        res = jnp.where(lanes == q * 4 + t, s, res)
                    return res

                res = lax.fori_loop(0, 4, quad,
                                    jnp.zeros((16,), jnp.float32))
                obuf[pl.ds(c * _CH + g * 16, 16)] = res + bias
                return _

            lax.fori_loop(0, _CH // 16, row_group, jnp.int32(0))
        return carry

    lax.fori_loop(0, nch // 2, chunk_pair, jnp.int32(0))
    pltpu.make_async_copy(d_hbm.at[b, pl.ds(r0, _CH)],
                          dbuf.at[0], sems[0]).wait()

    pltpu.sync_copy(obuf, out_hbm.at[bsc, pl.ds(r0, rows_per_w)])


def _sc_matvec(D, c3, nbatch):
    B, L, K = D.shape
    rows_per_w = (nbatch * L) // 32
    mesh = plsc.VectorSubcoreMesh(core_axis_name="c", subcore_axis_name="s")
    kern = functools.partial(
        pl.kernel,
        out_type=jax.ShapeDtypeStruct((nbatch, L), jnp.float32),
        mesh=mesh,
        scratch_types=[
            pltpu.VMEM((2, _CH, K), jnp.float32),
            pltpu.VMEM((K,), jnp.float32),
            pltpu.VMEM((16,), jnp.float32),
            pltpu.VMEM((rows_per_w,), jnp.float32),
            pltpu.SemaphoreType.DMA,
            pltpu.SemaphoreType.DMA,
        ],
    )(functools.partial(_sc_body, nbatch, rows_per_w))
    return kern(D, c3)


def kernel(dict, coef):
    D = dict
    B, L, K = D.shape
    c3 = jnp.pad(coef, ((0, 0), (0, _KP - coef.shape[1]))).reshape(B, 1, _KP)

    nb_tc = B - _S
    out_sc = _sc_matvec(D, c3, _S)
    out_tc = _tc_matvec(D, c3, nb_tc)
    return jnp.concatenate([out_tc, out_sc.reshape(_S, L, 1)], axis=0)


# speedup vs baseline: 1.7917x; 1.7917x over previous
import jax
import jax.numpy as jnp
from jax.experimental import pallas as pl

_BB = 4
_HB = _BB // 2


def _matvec_body(d0_ref, d1_ref, c_ref, o_ref):
    for j, d_ref in enumerate((d0_ref, d1_ref)):
        for i in range(_HB):
            bi = j * _HB + i
            d = d_ref[i]
            w = c_ref[bi, :, 0:1024]
            acc = jnp.sum(d * w, axis=1)
            o_ref[bi] = acc[None, :] + c_ref[bi, 0, 1024]


def kernel(dict, coef):
    D = dict
    B, L, K = D.shape
    KC = coef.shape[1]
    c3 = coef.reshape(B, 1, KC)

    out = pl.pallas_call(
        _matvec_body,
        grid=(B // _BB,),
        in_specs=[
            pl.BlockSpec((_HB, L, K), lambda b: (2 * b, 0, 0)),
            pl.BlockSpec((_HB, L, K), lambda b: (2 * b + 1, 0, 0)),
            pl.BlockSpec((_BB, 1, KC), lambda b: (b, 0, 0)),
        ],
        out_specs=pl.BlockSpec((_BB, 1, L), lambda b: (b, 0, 0)),
        out_shape=jax.ShapeDtypeStruct((B, 1, L), jnp.float32),
    )(D, D, c3)
    return out.reshape(B, L, 1)


# speedup vs baseline: 1.7931x; 1.0007x over previous
import jax
import jax.numpy as jnp
from jax.experimental import pallas as pl

_BB = 8
_HB = _BB // 2
_RS = 512


def _matvec_body(d0_ref, d1_ref, c_ref, o_ref):
    gb = pl.program_id(0) * _BB
    cv = c_ref[pl.ds(gb, _BB), :]
    for j, d_ref in enumerate((d0_ref, d1_ref)):
        for i in range(_HB):
            bi = j * _HB + i
            d = d_ref[i]
            w = cv[bi:bi + 1, 0:1024]
            acc = jnp.sum(d * w, axis=1)
            o_ref[bi] = acc + cv[bi, 1024]


def kernel(dict, coef):
    D = dict
    B, L, K = D.shape
    KC = coef.shape[1]

    out = pl.pallas_call(
        _matvec_body,
        grid=(B // _BB, L // _RS),
        in_specs=[
            pl.BlockSpec((_HB, _RS, K), lambda b, r: (2 * b, r, 0)),
            pl.BlockSpec((_HB, _RS, K), lambda b, r: (2 * b + 1, r, 0)),
            pl.BlockSpec((B, KC), lambda b, r: (0, 0)),
        ],
        out_specs=pl.BlockSpec((_BB, _RS), lambda b, r: (b, r)),
        out_shape=jax.ShapeDtypeStruct((B, L), jnp.float32),
    )(D, D, coef)
    return out.reshape(B, L, 1)


# speedup vs baseline: 1.8277x; 1.0193x over previous
import jax
import jax.numpy as jnp
from jax.experimental import pallas as pl

_BB = 8
_HB = _BB // 2
_RS = 512


def _matvec_body(d0_ref, d1_ref, c_ref, o_ref):
    gb = pl.program_id(0) * _BB
    cv = c_ref[pl.ds(gb, _BB), :]
    for j, d_ref in enumerate((d0_ref, d1_ref)):
        for i in range(_HB):
            bi = j * _HB + i
            d = d_ref[i]
            w = cv[bi:bi + 1, 0:1024]
            acc = jnp.sum(d * w, axis=1)
            o_ref[bi, 0] = acc + cv[bi, 1024]


def kernel(dict, coef):
    D = dict
    B, L, K = D.shape
    KC = coef.shape[1]

    out = pl.pallas_call(
        _matvec_body,
        grid=(B // _BB, L // _RS),
        in_specs=[
            pl.BlockSpec((_HB, _RS, K), lambda b, r: (2 * b, r, 0)),
            pl.BlockSpec((_HB, _RS, K), lambda b, r: (2 * b + 1, r, 0)),
            pl.BlockSpec((B, KC), lambda b, r: (0, 0)),
        ],
        out_specs=pl.BlockSpec((_BB, 1, _RS), lambda b, r: (b, 0, r)),
        out_shape=jax.ShapeDtypeStruct((B, 1, L), jnp.float32),
    )(D, D, coef)
    return out.reshape(B, L, 1)


# speedup vs baseline: 1.8324x; 1.0026x over previous
import jax
import jax.numpy as jnp
from jax.experimental import pallas as pl

_BB = 8
_NS = 4
_HB = _BB // _NS
_RS = 512


def _matvec_body(d0_ref, d1_ref, d2_ref, d3_ref, c_ref, o_ref):
    gb = pl.program_id(0) * _BB
    cv = c_ref[pl.ds(gb, _BB), :]
    for j, d_ref in enumerate((d0_ref, d1_ref, d2_ref, d3_ref)):
        for i in range(_HB):
            bi = j * _HB + i
            d = d_ref[i]
            w = cv[bi:bi + 1, 0:1024]
            acc = jnp.sum(d * w, axis=1)
            o_ref[bi, 0] = acc + cv[bi, 1024]


def kernel(dict, coef):
    D = dict
    B, L, K = D.shape
    KC = coef.shape[1]

    dspec = [
        pl.BlockSpec((_HB, _RS, K),
                     (lambda s: (lambda b, r: (_NS * b + s, r, 0)))(s))
        for s in range(_NS)
    ]
    out = pl.pallas_call(
        _matvec_body,
        grid=(B // _BB, L // _RS),
        in_specs=dspec + [pl.BlockSpec((B, KC), lambda b, r: (0, 0))],
        out_specs=pl.BlockSpec((_BB, 1, _RS), lambda b, r: (b, 0, r)),
        out_shape=jax.ShapeDtypeStruct((B, 1, L), jnp.float32),
    )(D, D, D, D, coef)
    return out.reshape(B, L, 1)
